# baseline (speedup 1.0000x reference)
import functools

import jax
import jax.numpy as jnp
from jax import lax
from jax.experimental import pallas as pl
from jax.experimental.pallas import tpu as pltpu
from jax.experimental.pallas import tpu_sc as plsc

N = 10000
E = 320000
D = 128

NC = 2
NS = 16
NW = NC * NS
EPW = E // NW
BE = 80
NB = EPW // BE
BE2 = 96
NB2 = 105
EPW2 = BE2 * NB2
E2 = NW * EPW2
NP = 10240
RPT = NP // NS
GR = NP // D

_mesh = plsc.VectorSubcoreMesh(core_axis_name="c", subcore_axis_name="s")


@functools.partial(
    pl.kernel,
    out_type=(
        jax.ShapeDtypeStruct((NC * NP,), jnp.float32),
        jax.ShapeDtypeStruct((NC * NP,), jnp.float32),
    ),
    mesh=_mesh,
    compiler_params=pltpu.CompilerParams(needs_layout_passes=False),
    scratch_types=(
        pltpu.VMEM((EPW,), jnp.int32),
        pltpu.VMEM((EPW,), jnp.int32),
        pltpu.VMEM((NP,), jnp.float32),
        pltpu.VMEM((NP,), jnp.float32),
        pltpu.VMEM((RPT,), jnp.float32),
        pltpu.VMEM((RPT,), jnp.float32),
        pltpu.VMEM_SHARED((NS, NP), jnp.float32),
        pltpu.VMEM_SHARED((NS, NP), jnp.float32),
    ),
)
def _sc_degrees(srcf_hbm, dstf_hbm, dout_hbm, din_hbm,
                srcv, dstv, co_v, ci_v, tmp_v, acc_v, stage_o, stage_i):
    cid = lax.axis_index("c")
    sid = lax.axis_index("s")
    wid = cid * NS + sid
    pltpu.sync_copy(srcf_hbm.at[pl.ds(wid * EPW, EPW)], srcv)
    pltpu.sync_copy(dstf_hbm.at[pl.ds(wid * EPW, EPW)], dstv)

    zeros16 = jnp.zeros((16,), jnp.float32)
    ones16 = jnp.ones((16,), jnp.float32)

    def zbody(i, carry):
        co_v[pl.ds(i * 16, 16)] = zeros16
        ci_v[pl.ds(i * 16, 16)] = zeros16
        return carry

    lax.fori_loop(0, NP // 16, zbody, 0)

    def body(i, carry):
        plsc.addupdate_scatter(co_v, [srcv[pl.ds(i * 16, 16)]], ones16)
        plsc.addupdate_scatter(ci_v, [dstv[pl.ds(i * 16, 16)]], ones16)
        return carry

    lax.fori_loop(0, EPW // 16, body, 0)
    pltpu.sync_copy(co_v, stage_o.at[sid])
    pltpu.sync_copy(ci_v, stage_i.at[sid])
    plsc.subcore_barrier()
    base = cid * NP + sid * RPT
    for stage, out_hbm in ((stage_o, dout_hbm), (stage_i, din_hbm)):
        pltpu.sync_copy(stage.at[0, pl.ds(sid * RPT, RPT)], acc_v)
        for t in range(1, NS):
            pltpu.sync_copy(stage.at[t, pl.ds(sid * RPT, RPT)], tmp_v)

            def abody(k, carry):
                acc_v[pl.ds(k * 16, 16)] = (acc_v[pl.ds(k * 16, 16)]
                                            + tmp_v[pl.ds(k * 16, 16)])
                return carry

            lax.fori_loop(0, RPT // 16, abody, 0)
        pltpu.sync_copy(acc_v, out_hbm.at[pl.ds(base, RPT)])


@functools.partial(
    pl.kernel,
    out_type=jax.ShapeDtypeStruct((NC * NP, D), jnp.float32),
    mesh=_mesh,
    scratch_types=(
        pltpu.VMEM((EPW2,), jnp.int32),
        pltpu.VMEM((NB2, BE2), jnp.int32),
        pltpu.VMEM((BE2, D), jnp.float32),
        pltpu.VMEM((BE2, D), jnp.float32),
        pltpu.VMEM_SHARED((NP, D), jnp.float32),
        pltpu.SemaphoreType.DMA,
        pltpu.SemaphoreType.DMA,
    ),
)
def _sc_spmm(xp_hbm, srcf_hbm, dst_hbm, zeros_hbm, out_hbm,
             src_v, dst_v, rows0, rows1, agg_sh, sem0, sem1):
    cid = lax.axis_index("c")
    sid = lax.axis_index("s")
    wid = cid * NS + sid
    pltpu.sync_copy(zeros_hbm.at[pl.ds(sid * RPT, RPT)],
                    agg_sh.at[pl.ds(sid * RPT, RPT)])
    pltpu.sync_copy(srcf_hbm.at[pl.ds(wid * EPW2, EPW2)], src_v)
    pltpu.sync_copy(dst_hbm.at[wid], dst_v)
    plsc.subcore_barrier()

    def _gather(j, rows, sem):
        return pltpu.async_copy(xp_hbm.at[src_v.at[pl.ds(j * BE2, BE2)]],
                                rows, sem)

    def _gwait(j, rows, sem):
        pltpu.make_async_copy(xp_hbm.at[src_v.at[pl.ds(j * BE2, BE2)]],
                              rows, sem).wait()

    _gather(0, rows0, sem0)

    def body(g, carry):
        j0 = 2 * g
        _gather(j0 + 1, rows1, sem1)
        _gwait(j0, rows0, sem0)
        pltpu.sync_copy(rows0, agg_sh.at[dst_v.at[j0]], add=True)

        @pl.when(j0 + 2 < NB2)
        def _next_even():
            _gather(j0 + 2, rows0, sem0)

        _gwait(j0 + 1, rows1, sem1)
        pltpu.sync_copy(rows1, agg_sh.at[dst_v.at[j0 + 1]], add=True)
        return carry

    lax.fori_loop(0, NB2 // 2, body, 0)
    _gwait(NB2 - 1, rows0, sem0)
    pltpu.sync_copy(rows0, agg_sh.at[dst_v.at[NB2 - 1]], add=True)
    plsc.subcore_barrier()
    base = cid * NP + sid * RPT
    pltpu.sync_copy(agg_sh.at[pl.ds(sid * RPT, RPT)],
                    out_hbm.at[pl.ds(base, RPT)])


_RB = 1000


def _tc_prescale_body(x_ref, d0_ref, d1_ref, o_ref):
    deg = d0_ref[...] + d1_ref[...]
    s = lax.rsqrt(jnp.maximum(deg, 1.0))
    o_ref[...] = x_ref[...] * s


def _tc_prescale(x, d0, d1):
    return pl.pallas_call(
        _tc_prescale_body,
        grid=(N // _RB,),
        in_specs=[
            pl.BlockSpec((_RB, D), lambda i: (i, 0)),
            pl.BlockSpec((_RB, 1), lambda i: (i, 0)),
            pl.BlockSpec((_RB, 1), lambda i: (i, 0)),
        ],
        out_specs=pl.BlockSpec((_RB, D), lambda i: (i, 0)),
        out_shape=jax.ShapeDtypeStruct((N, D), jnp.float32),
    )(x, d0, d1)


def _tc_finish_body(a0_ref, a1_ref, d0_ref, d1_ref, w_ref, o_ref):
    deg = d0_ref[...] + d1_ref[...]
    s = lax.rsqrt(jnp.maximum(deg, 1.0))
    agg = (a0_ref[...] + a1_ref[...]) * s
    o_ref[...] = jnp.tanh(jnp.dot(agg, w_ref[...],
                                  preferred_element_type=jnp.float32))


def _tc_finish(aggp, d0, d1, W):
    return pl.pallas_call(
        _tc_finish_body,
        grid=(N // _RB,),
        in_specs=[
            pl.BlockSpec((_RB, D), lambda i: (i, 0)),
            pl.BlockSpec((_RB, D), lambda i: (i, 0)),
            pl.BlockSpec((_RB, 1), lambda i: (i, 0)),
            pl.BlockSpec((_RB, 1), lambda i: (i, 0)),
            pl.BlockSpec((D, D), lambda i: (0, 0)),
        ],
        out_specs=pl.BlockSpec((_RB, D), lambda i: (i, 0)),
        out_shape=jax.ShapeDtypeStruct((N, D), jnp.float32),
    )(aggp[:N], aggp[NP:NP + N], d0, d1, W)


def kernel(x, edge_index, W):
    srcf = edge_index[0]
    dstf = edge_index[1]
    srcp = jnp.concatenate([srcf, jnp.zeros((E2 - E,), jnp.int32)])
    dstp3 = jnp.concatenate(
        [dstf, jnp.full((E2 - E,), NP - 1, jnp.int32)]).reshape(NW, NB2, BE2)
    zeros_nd = jnp.zeros((NP, D), jnp.float32)

    dout, din = _sc_degrees(srcf, dstf)
    do0 = dout[:N, None]
    do1 = dout[NP:NP + N, None]
    di0 = din[:N, None]
    di1 = din[NP:NP + N, None]
    xp = _tc_prescale(x, do0, do1)
    xpp = jnp.concatenate([xp, jnp.zeros((NP - N, D), jnp.float32)])
    aggp = _sc_spmm(xpp, srcp, dstp3, zeros_nd)
    return _tc_finish(aggp, di0, di1, W)


# speedup vs baseline: 1.5402x; 1.5402x over previous
import functools

import jax
import jax.numpy as jnp
from jax import lax
from jax.experimental import pallas as pl
from jax.experimental.pallas import tpu as pltpu
from jax.experimental.pallas import tpu_sc as plsc

N = 10000
E = 320000
D = 128

NC = 2
NS = 16
NW = NC * NS
EPW = E // NW
BE = 80
NB = EPW // BE
NP = 10240
RPT = NP // NS
GR = NP // D

_mesh = plsc.VectorSubcoreMesh(core_axis_name="c", subcore_axis_name="s")


@functools.partial(
    pl.kernel,
    out_type=(
        jax.ShapeDtypeStruct((NC * NP,), jnp.float32),
        jax.ShapeDtypeStruct((NC * NP,), jnp.float32),
    ),
    mesh=_mesh,
    compiler_params=pltpu.CompilerParams(needs_layout_passes=False),
    scratch_types=(
        pltpu.VMEM((EPW,), jnp.int32),
        pltpu.VMEM((EPW,), jnp.int32),
        pltpu.VMEM((NP,), jnp.float32),
        pltpu.VMEM((NP,), jnp.float32),
        pltpu.VMEM((RPT,), jnp.float32),
        pltpu.VMEM((RPT,), jnp.float32),
        pltpu.VMEM_SHARED((NS, NP), jnp.float32),
        pltpu.VMEM_SHARED((NS, NP), jnp.float32),
    ),
)
def _sc_degrees(srcf_hbm, dstf_hbm, dout_hbm, din_hbm,
                srcv, dstv, co_v, ci_v, tmp_v, acc_v, stage_o, stage_i):
    cid = lax.axis_index("c")
    sid = lax.axis_index("s")
    wid = cid * NS + sid
    pltpu.sync_copy(srcf_hbm.at[pl.ds(wid * EPW, EPW)], srcv)
    pltpu.sync_copy(dstf_hbm.at[pl.ds(wid * EPW, EPW)], dstv)

    zeros16 = jnp.zeros((16,), jnp.float32)
    ones16 = jnp.ones((16,), jnp.float32)

    def zbody(i, carry):
        co_v[pl.ds(i * 16, 16)] = zeros16
        ci_v[pl.ds(i * 16, 16)] = zeros16
        return carry

    lax.fori_loop(0, NP // 16, zbody, 0)

    def body(i, carry):
        plsc.addupdate_scatter(co_v, [srcv[pl.ds(i * 16, 16)]], ones16)
        plsc.addupdate_scatter(ci_v, [dstv[pl.ds(i * 16, 16)]], ones16)
        return carry

    lax.fori_loop(0, EPW // 16, body, 0)
    pltpu.sync_copy(co_v, stage_o.at[sid])
    pltpu.sync_copy(ci_v, stage_i.at[sid])
    plsc.subcore_barrier()
    base = cid * NP + sid * RPT
    for stage, out_hbm in ((stage_o, dout_hbm), (stage_i, din_hbm)):
        pltpu.sync_copy(stage.at[0, pl.ds(sid * RPT, RPT)], acc_v)
        for t in range(1, NS):
            pltpu.sync_copy(stage.at[t, pl.ds(sid * RPT, RPT)], tmp_v)

            def abody(k, carry):
                acc_v[pl.ds(k * 16, 16)] = (acc_v[pl.ds(k * 16, 16)]
                                            + tmp_v[pl.ds(k * 16, 16)])
                return carry

            lax.fori_loop(0, RPT // 16, abody, 0)
        pltpu.sync_copy(acc_v, out_hbm.at[pl.ds(base, RPT)])


@functools.partial(
    pl.kernel,
    out_type=jax.ShapeDtypeStruct((NC * NP, D), jnp.float32),
    mesh=_mesh,
    scratch_types=(
        pltpu.VMEM((EPW,), jnp.int32),
        pltpu.VMEM((NB, BE), jnp.int32),
        pltpu.VMEM((BE, D), jnp.float32),
        pltpu.VMEM((BE, D), jnp.float32),
        pltpu.VMEM_SHARED((NP, D), jnp.float32),
        pltpu.SemaphoreType.DMA,
        pltpu.SemaphoreType.DMA,
    ),
)
def _sc_spmm(xp_hbm, srcf_hbm, dst_hbm, zeros_hbm, out_hbm,
             src_v, dst_v, rows0, rows1, agg_sh, sem0, sem1):
    cid = lax.axis_index("c")
    sid = lax.axis_index("s")
    wid = cid * NS + sid
    pltpu.sync_copy(zeros_hbm.at[pl.ds(sid * RPT, RPT)],
                    agg_sh.at[pl.ds(sid * RPT, RPT)])
    pltpu.sync_copy(srcf_hbm.at[pl.ds(wid * EPW, EPW)], src_v)
    pltpu.sync_copy(dst_hbm.at[wid], dst_v)
    plsc.subcore_barrier()

    def _gather(j, rows, sem):
        return pltpu.async_copy(xp_hbm.at[src_v.at[pl.ds(j * BE, BE)]],
                                rows, sem)

    def _gwait(j, rows, sem):
        pltpu.make_async_copy(xp_hbm.at[src_v.at[pl.ds(j * BE, BE)]],
                              rows, sem).wait()

    _gather(0, rows0, sem0)

    def body(g, carry):
        j0 = 2 * g
        _gather(j0 + 1, rows1, sem1)
        _gwait(j0, rows0, sem0)
        pltpu.sync_copy(rows0, agg_sh.at[dst_v.at[j0]], add=True)

        @pl.when(j0 + 2 < NB)
        def _next_even():
            _gather(j0 + 2, rows0, sem0)

        _gwait(j0 + 1, rows1, sem1)
        pltpu.sync_copy(rows1, agg_sh.at[dst_v.at[j0 + 1]], add=True)
        return carry

    lax.fori_loop(0, NB // 2, body, 0)
    _gwait(NB - 1, rows0, sem0)
    pltpu.sync_copy(rows0, agg_sh.at[dst_v.at[NB - 1]], add=True)
    plsc.subcore_barrier()
    base = cid * NP + sid * RPT
    pltpu.sync_copy(agg_sh.at[pl.ds(sid * RPT, RPT)],
                    out_hbm.at[pl.ds(base, RPT)])


_RB = 1000


def _tc_prescale_body(x_ref, d0_ref, d1_ref, o_ref):
    deg = d0_ref[...] + d1_ref[...]
    s = lax.rsqrt(jnp.maximum(deg, 1.0))
    o_ref[...] = x_ref[...] * s


def _tc_prescale(x, d0, d1):
    return pl.pallas_call(
        _tc_prescale_body,
        grid=(N // _RB,),
        in_specs=[
            pl.BlockSpec((_RB, D), lambda i: (i, 0)),
            pl.BlockSpec((_RB, 1), lambda i: (i, 0)),
            pl.BlockSpec((_RB, 1), lambda i: (i, 0)),
        ],
        out_specs=pl.BlockSpec((_RB, D), lambda i: (i, 0)),
        out_shape=jax.ShapeDtypeStruct((N, D), jnp.float32),
    )(x, d0, d1)


def _tc_finish_body(a0_ref, a1_ref, d0_ref, d1_ref, w_ref, o_ref):
    deg = d0_ref[...] + d1_ref[...]
    s = lax.rsqrt(jnp.maximum(deg, 1.0))
    agg = (a0_ref[...] + a1_ref[...]) * s
    o_ref[...] = jnp.tanh(jnp.dot(agg, w_ref[...],
                                  preferred_element_type=jnp.float32))


def _tc_finish(aggp, d0, d1, W):
    return pl.pallas_call(
        _tc_finish_body,
        grid=(N // _RB,),
        in_specs=[
            pl.BlockSpec((_RB, D), lambda i: (i, 0)),
            pl.BlockSpec((_RB, D), lambda i: (i, 0)),
            pl.BlockSpec((_RB, 1), lambda i: (i, 0)),
            pl.BlockSpec((_RB, 1), lambda i: (i, 0)),
            pl.BlockSpec((D, D), lambda i: (0, 0)),
        ],
        out_specs=pl.BlockSpec((_RB, D), lambda i: (i, 0)),
        out_shape=jax.ShapeDtypeStruct((N, D), jnp.float32),
    )(aggp[:N], aggp[NP:NP + N], d0, d1, W)


def kernel(x, edge_index, W):
    srcf = edge_index[0]
    dstf = edge_index[1]
    dst3 = dstf.reshape(NW, NB, BE)
    zeros_nd = jnp.zeros((NP, D), jnp.float32)

    dout, din = _sc_degrees(srcf, dstf)
    do0 = dout[:N, None]
    do1 = dout[NP:NP + N, None]
    di0 = din[:N, None]
    di1 = din[NP:NP + N, None]
    xp = _tc_prescale(x, do0, do1)
    aggp = _sc_spmm(xp, srcf, dst3, zeros_nd)
    return _tc_finish(aggp, di0, di1, W)


# speedup vs baseline: 1.5420x; 1.0012x over previous
import functools

import jax
import jax.numpy as jnp
from jax import lax
from jax.experimental import pallas as pl
from jax.experimental.pallas import tpu as pltpu
from jax.experimental.pallas import tpu_sc as plsc

N = 10000
E = 320000
D = 128

NC = 2
NS = 16
NW = NC * NS
EPW = E // NW
BE = 80
NB = EPW // BE
NP = 10240
RPT = NP // NS

_mesh = plsc.VectorSubcoreMesh(core_axis_name="c", subcore_axis_name="s")


@functools.partial(
    pl.kernel,
    out_type=(
        jax.ShapeDtypeStruct((NC * NP,), jnp.float32),
        jax.ShapeDtypeStruct((NC * NP,), jnp.float32),
    ),
    mesh=_mesh,
    compiler_params=pltpu.CompilerParams(needs_layout_passes=False),
    scratch_types=(
        pltpu.VMEM((EPW,), jnp.int32),
        pltpu.VMEM((EPW,), jnp.int32),
        pltpu.VMEM((NP,), jnp.float32),
        pltpu.VMEM((NP,), jnp.float32),
        pltpu.VMEM((RPT,), jnp.float32),
        pltpu.VMEM((RPT,), jnp.float32),
        pltpu.VMEM_SHARED((NS, NP), jnp.float32),
        pltpu.VMEM_SHARED((NS, NP), jnp.float32),
    ),
)
def _sc_degrees(srcf_hbm, dstf_hbm, dout_hbm, din_hbm,
                srcv, dstv, co_v, ci_v, tmp_v, acc_v, stage_o, stage_i):
    cid = lax.axis_index("c")
    sid = lax.axis_index("s")
    wid = cid * NS + sid
    pltpu.sync_copy(srcf_hbm.at[pl.ds(wid * EPW, EPW)], srcv)
    pltpu.sync_copy(dstf_hbm.at[pl.ds(wid * EPW, EPW)], dstv)

    zeros16 = jnp.zeros((16,), jnp.float32)
    ones16 = jnp.ones((16,), jnp.float32)

    def zbody(i, carry):
        co_v[pl.ds(i * 16, 16)] = zeros16
        ci_v[pl.ds(i * 16, 16)] = zeros16
        return carry

    lax.fori_loop(0, NP // 16, zbody, 0)

    def body(i, carry):
        plsc.addupdate_scatter(co_v, [srcv[pl.ds(i * 16, 16)]], ones16)
        plsc.addupdate_scatter(ci_v, [dstv[pl.ds(i * 16, 16)]], ones16)
        return carry

    lax.fori_loop(0, EPW // 16, body, 0)
    pltpu.sync_copy(co_v, stage_o.at[sid])
    pltpu.sync_copy(ci_v, stage_i.at[sid])
    plsc.subcore_barrier()
    base = cid * NP + sid * RPT
    for stage, out_hbm in ((stage_o, dout_hbm), (stage_i, din_hbm)):
        pltpu.sync_copy(stage.at[0, pl.ds(sid * RPT, RPT)], acc_v)
        for t in range(1, NS):
            pltpu.sync_copy(stage.at[t, pl.ds(sid * RPT, RPT)], tmp_v)

            def abody(k, carry):
                acc_v[pl.ds(k * 16, 16)] = (acc_v[pl.ds(k * 16, 16)]
                                            + tmp_v[pl.ds(k * 16, 16)])
                return carry

            lax.fori_loop(0, RPT // 16, abody, 0)
        pltpu.sync_copy(acc_v, out_hbm.at[pl.ds(base, RPT)])


@functools.partial(
    pl.kernel,
    out_type=jax.ShapeDtypeStruct((NC * NP, D), jnp.float32),
    mesh=_mesh,
    scratch_types=(
        pltpu.VMEM((EPW,), jnp.int32),
        pltpu.VMEM((NB, BE), jnp.int32),
        pltpu.VMEM((BE, D), jnp.float32),
        pltpu.VMEM((BE, D), jnp.float32),
        pltpu.VMEM_SHARED((NP, D), jnp.float32),
        pltpu.SemaphoreType.DMA,
        pltpu.SemaphoreType.DMA,
    ),
)
def _sc_spmm(xp_hbm, srcf_hbm, dst_hbm, zeros_hbm, out_hbm,
             src_v, dst_v, rows0, rows1, agg_sh, sem0, sem1):
    cid = lax.axis_index("c")
    sid = lax.axis_index("s")
    wid = cid * NS + sid
    pltpu.sync_copy(zeros_hbm.at[pl.ds(sid * RPT, RPT)],
                    agg_sh.at[pl.ds(sid * RPT, RPT)])
    pltpu.sync_copy(srcf_hbm.at[pl.ds(wid * EPW, EPW)], src_v)
    pltpu.sync_copy(dst_hbm.at[wid], dst_v)
    plsc.subcore_barrier()

    def _gather(j, rows, sem):
        return pltpu.async_copy(xp_hbm.at[src_v.at[pl.ds(j * BE, BE)]],
                                rows, sem)

    def _gwait(j, rows, sem):
        pltpu.make_async_copy(xp_hbm.at[src_v.at[pl.ds(j * BE, BE)]],
                              rows, sem).wait()

    _gather(0, rows0, sem0)

    def body(g, carry):
        j0 = 2 * g
        _gather(j0 + 1, rows1, sem1)
        _gwait(j0, rows0, sem0)
        pltpu.sync_copy(rows0, agg_sh.at[dst_v.at[j0]], add=True)

        @pl.when(j0 + 2 < NB)
        def _next_even():
            _gather(j0 + 2, rows0, sem0)

        _gwait(j0 + 1, rows1, sem1)
        pltpu.sync_copy(rows1, agg_sh.at[dst_v.at[j0 + 1]], add=True)
        return carry

    lax.fori_loop(0, NB // 2, body, 0)
    _gwait(NB - 1, rows0, sem0)
    pltpu.sync_copy(rows0, agg_sh.at[dst_v.at[NB - 1]], add=True)
    plsc.subcore_barrier()
    base = cid * NP + sid * RPT
    pltpu.sync_copy(agg_sh.at[pl.ds(sid * RPT, RPT)],
                    out_hbm.at[pl.ds(base, RPT)])


_RB = 1000


def _tc_prescale_body(x_ref, d0_ref, d1_ref, o_ref):
    deg = d0_ref[...] + d1_ref[...]
    s = lax.rsqrt(jnp.maximum(deg, 1.0))
    o_ref[...] = x_ref[...] * s


def _tc_prescale(x, d0, d1):
    return pl.pallas_call(
        _tc_prescale_body,
        grid=(N // _RB,),
        in_specs=[
            pl.BlockSpec((_RB, D), lambda i: (i, 0)),
            pl.BlockSpec((_RB, 1), lambda i: (i, 0)),
            pl.BlockSpec((_RB, 1), lambda i: (i, 0)),
        ],
        out_specs=pl.BlockSpec((_RB, D), lambda i: (i, 0)),
        out_shape=jax.ShapeDtypeStruct((N, D), jnp.float32),
    )(x, d0, d1)


def _tc_finish_body(a0_ref, a1_ref, d0_ref, d1_ref, w_ref, o_ref):
    deg = d0_ref[...] + d1_ref[...]
    s = lax.rsqrt(jnp.maximum(deg, 1.0))
    agg = (a0_ref[...] + a1_ref[...]) * s
    o_ref[...] = jnp.tanh(jnp.dot(agg, w_ref[...],
                                  preferred_element_type=jnp.float32))


def _tc_finish(aggp, d0, d1, W):
    return pl.pallas_call(
        _tc_finish_body,
        grid=(N // _RB,),
        in_specs=[
            pl.BlockSpec((_RB, D), lambda i: (i, 0)),
            pl.BlockSpec((_RB, D), lambda i: (i, 0)),
            pl.BlockSpec((_RB, 1), lambda i: (i, 0)),
            pl.BlockSpec((_RB, 1), lambda i: (i, 0)),
            pl.BlockSpec((D, D), lambda i: (0, 0)),
        ],
        out_specs=pl.BlockSpec((_RB, D), lambda i: (i, 0)),
        out_shape=jax.ShapeDtypeStruct((N, D), jnp.float32),
    )(aggp[:N], aggp[NP:NP + N], d0, d1, W)


def kernel(x, edge_index, W):
    srcf = edge_index[0]
    dstf = edge_index[1]
    dst3 = dstf.reshape(NW, NB, BE)
    zeros_nd = jnp.zeros((NP, D), jnp.float32)

    dout, din = _sc_degrees(srcf, dstf)
    do0 = dout[:N, None]
    do1 = dout[NP:NP + N, None]
    di0 = din[:N, None]
    di1 = din[NP:NP + N, None]
    xp = _tc_prescale(x, do0, do1)
    aggp = _sc_spmm(xp, srcf, dst3, zeros_nd)
    return _tc_finish(aggp, di0, di1, W)
